# baseline (speedup 1.0000x reference)
import functools

import jax
import jax.numpy as jnp
from jax import lax
from jax.experimental import pallas as pl
from jax.experimental.pallas import tpu as pltpu
from jax.experimental.pallas import tpu_sc as plsc

N = 10000
E = 320000
D = 128
T = 10
STEPS = 8
B = 64
L = 128
DS = 512
C = 14

NC = 2
NS = 16
NW = NC * NS
K = 128
NCHUNK = 79
PW = NCHUNK * K
E_PAD = NW * PW
NACC = N + 112
RPS = NACC // NS

BN = 2000
NBLK = N // BN
BN1 = 1000
NBLK1 = N // BN1
BL = 32


def _build_sc_agg():
    mesh = plsc.VectorSubcoreMesh(core_axis_name="c", subcore_axis_name="s")

    @functools.partial(
        pl.kernel,
        mesh=mesh,
        out_type=jax.ShapeDtypeStruct((NC, NACC, D), jnp.float32),
        scratch_types=[
            pltpu.VMEM_SHARED((NACC, D), jnp.float32),
            pltpu.VMEM((K,), jnp.int32),
            pltpu.VMEM((K,), jnp.int32),
            pltpu.VMEM((K, D), jnp.float32),
            pltpu.SemaphoreType.DMA,
        ],
    )
    def sc_agg(trans_hbm, gidx_hbm, dst_hbm, zeros_hbm, out_hbm,
               acc, idxc, dstc, rows, sem):
        c = lax.axis_index("c")
        s = lax.axis_index("s")
        wid = s * NC + c
        row0 = s * RPS
        pltpu.sync_copy(zeros_hbm.at[pl.ds(row0, RPS)],
                        acc.at[pl.ds(row0, RPS)])
        plsc.subcore_barrier()
        base = wid * PW

        def body(j, carry):
            off = base + j * K
            pltpu.sync_copy(gidx_hbm.at[pl.ds(off, K)], idxc)
            pltpu.sync_copy(dst_hbm.at[pl.ds(off, K)], dstc)
            pltpu.async_copy(trans_hbm.at[idxc], rows, sem).wait()
            pltpu.sync_copy(rows, acc.at[dstc], add=True)
            return carry

        lax.fori_loop(0, NCHUNK, body, 0)
        plsc.subcore_barrier()
        pltpu.sync_copy(acc.at[pl.ds(row0, RPS)],
                        out_hbm.at[c, pl.ds(row0, RPS)])

    return sc_agg


def _transform_body(h_ref, w_ref, b_ref, out_ref):
    out_ref[0] = (
        jnp.dot(h_ref[...], w_ref[0], preferred_element_type=jnp.float32)
        + b_ref[0]
    )


def _build_transform(interpret=False):
    return pl.pallas_call(
        _transform_body,
        grid=(T, NBLK),
        in_specs=[
            pl.BlockSpec((BN, D), lambda t, n: (n, 0)),
            pl.BlockSpec((1, D, D), lambda t, n: (t, 0, 0)),
            pl.BlockSpec((1, 1, D), lambda t, n: (t, 0, 0)),
        ],
        out_specs=pl.BlockSpec((1, BN, D), lambda t, n: (t, n, 0)),
        out_shape=jax.ShapeDtypeStruct((T, N, D), jnp.float32),
        interpret=interpret,
    )


def _gru_body(part_ref, h_ref, wih_ref, whh_ref, bih_ref, bhh_ref, out_ref):
    agg = part_ref[0] + part_ref[1]
    h = h_ref[...]
    gi = jnp.dot(agg, wih_ref[...], preferred_element_type=jnp.float32) + bih_ref[...]
    gh = jnp.dot(h, whh_ref[...], preferred_element_type=jnp.float32) + bhh_ref[...]
    r = jax.nn.sigmoid(gi[:, 0:D] + gh[:, 0:D])
    z = jax.nn.sigmoid(gi[:, D:2 * D] + gh[:, D:2 * D])
    n = jnp.tanh(gi[:, 2 * D:3 * D] + r * gh[:, 2 * D:3 * D])
    out_ref[...] = (1.0 - z) * n + z * h


def _build_gru(interpret=False):
    return pl.pallas_call(
        _gru_body,
        grid=(NBLK,),
        in_specs=[
            pl.BlockSpec((NC, BN, D), lambda n: (0, n, 0)),
            pl.BlockSpec((BN, D), lambda n: (n, 0)),
            pl.BlockSpec((D, 3 * D), lambda n: (0, 0)),
            pl.BlockSpec((D, 3 * D), lambda n: (0, 0)),
            pl.BlockSpec((1, 3 * D), lambda n: (0, 0)),
            pl.BlockSpec((1, 3 * D), lambda n: (0, 0)),
        ],
        out_specs=pl.BlockSpec((BN, D), lambda n: (n, 0)),
        out_shape=jax.ShapeDtypeStruct((N, D), jnp.float32),
        interpret=interpret,
    )


def _gsum_body(h_ref, x_ref, gid_ref, out_ref):
    i = pl.program_id(0)
    gids = gid_ref[0, 0, :]
    b_iota = lax.broadcasted_iota(jnp.int32, (B, BN1), 0)
    onehot = (gids[None, :] == b_iota).astype(jnp.float32)
    feat = jnp.concatenate(
        [h_ref[...], x_ref[...], jnp.ones((BN1, D), jnp.float32)], axis=1)
    contrib = jnp.dot(onehot, feat, preferred_element_type=jnp.float32)

    @pl.when(i == 0)
    def _():
        out_ref[...] = jnp.zeros_like(out_ref)

    out_ref[...] += contrib


def _build_gsum(interpret=False):
    return pl.pallas_call(
        _gsum_body,
        grid=(NBLK1,),
        in_specs=[
            pl.BlockSpec((BN1, D), lambda n: (n, 0)),
            pl.BlockSpec((BN1, D), lambda n: (n, 0)),
            pl.BlockSpec((1, 1, BN1), lambda n: (n, 0, 0)),
        ],
        out_specs=pl.BlockSpec((B, 3 * D), lambda n: (0, 0)),
        out_shape=jax.ShapeDtypeStruct((B, 3 * D), jnp.float32),
        interpret=interpret,
    )


def _seqsum_body(seq_ref, out_ref):
    i = pl.program_id(0)

    @pl.when(i == 0)
    def _():
        out_ref[...] = jnp.zeros_like(out_ref)

    out_ref[...] += jnp.sum(seq_ref[...], axis=1)


def _build_seqsum(interpret=False):
    return pl.pallas_call(
        _seqsum_body,
        grid=(L // BL,),
        in_specs=[pl.BlockSpec((B, BL, DS), lambda i: (0, i, 0))],
        out_specs=pl.BlockSpec((B, DS), lambda i: (0, 0)),
        out_shape=jax.ShapeDtypeStruct((B, DS), jnp.float32),
        interpret=interpret,
    )


def _final_body(gsum_ref, ssum_ref, wseq, bseq, wg0, bg0, wg1, bg1, wg2, bg2,
                ws0, bs0, ws1, bs1, ws2, bs2, out_ref):
    gs = gsum_ref[...]
    sums = gs[:, 0:2 * D]
    cnt = gs[:, 2 * D:2 * D + 1]
    gfeat = sums / jnp.maximum(cnt, 1.0)
    f = jnp.dot

    def relu(v):
        return jnp.maximum(v, 0.0)

    h1 = relu(f(gfeat, wg0[...], preferred_element_type=jnp.float32) + bg0[...])
    h2 = relu(f(h1, wg1[...], preferred_element_type=jnp.float32) + bg1[...])
    l1 = f(h2, wg2[...], preferred_element_type=jnp.float32) + bg2[...]
    smean = ssum_ref[...] * (1.0 / L)
    sf = relu(f(smean, wseq[...], preferred_element_type=jnp.float32) + bseq[...])
    s1 = relu(f(sf, ws0[...], preferred_element_type=jnp.float32) + bs0[...])
    s2 = relu(f(s1, ws1[...], preferred_element_type=jnp.float32) + bs1[...])
    l2 = f(s2, ws2[...], preferred_element_type=jnp.float32) + bs2[...]
    out_ref[...] = l1 + l2


def _build_final(interpret=False):
    full = lambda *shape: pl.BlockSpec(shape, lambda: (0,) * len(shape))
    return pl.pallas_call(
        _final_body,
        grid=(),
        in_specs=[
            full(B, 3 * D),
            full(B, DS),
            full(DS, 1024), full(1, 1024),
            full(2 * D, 128), full(1, 128),
            full(128, 64), full(1, 64),
            full(64, C), full(1, C),
            full(1024, 512), full(1, 512),
            full(512, 256), full(1, 256),
            full(256, C), full(1, C),
        ],
        out_specs=full(B, C),
        out_shape=jax.ShapeDtypeStruct((B, C), jnp.float32),
        interpret=interpret,
    )


def kernel(x, edge_index, edge_type, graph_ids, sequences,
           W_edge, b_edge, W_ih, W_hh, b_ih, b_hh, W_seq, b_seq,
           Wg0, bg0, Wg1, bg1, Wg2, bg2,
           Ws0, bs0, Ws1, bs1, Ws2, bs2):
    src = edge_index[0].astype(jnp.int32)
    dst = edge_index[1].astype(jnp.int32)
    et = edge_type.astype(jnp.int32)
    gidx = et * N + src
    order = jnp.argsort(gidx)
    gidx = gidx[order]
    dst = dst[order]
    pad = E_PAD - E
    gidx_p = jnp.concatenate([gidx, jnp.zeros((pad,), jnp.int32)])
    dst_p = jnp.concatenate([dst, jnp.full((pad,), N, jnp.int32)])
    zeros = jnp.zeros((NACC, D), jnp.float32)

    gids3 = graph_ids.astype(jnp.int32).reshape(NBLK1, 1, BN1)
    bih2 = b_ih.reshape(1, 3 * D)
    bhh2 = b_hh.reshape(1, 3 * D)

    transform = _build_transform()
    sc_agg = _build_sc_agg()
    gru = _build_gru()
    gsum_k = _build_gsum()
    seqsum_k = _build_seqsum()
    final_k = _build_final()

    h = x
    for _ in range(STEPS):
        trans = transform(h, W_edge, b_edge.reshape(T, 1, D)).reshape(T * N, D)
        parts = sc_agg(trans, gidx_p, dst_p, zeros)
        h = gru(parts, h, W_ih, W_hh, bih2, bhh2)

    gsum = gsum_k(h, x, gids3)
    ssum = seqsum_k(sequences)
    return final_k(
        gsum, ssum,
        W_seq, b_seq.reshape(1, 1024),
        Wg0, bg0.reshape(1, 128), Wg1, bg1.reshape(1, 64),
        Wg2, bg2.reshape(1, C),
        Ws0, bs0.reshape(1, 512), Ws1, bs1.reshape(1, 256),
        Ws2, bs2.reshape(1, C),
    )


# speedup vs baseline: 1.2894x; 1.2894x over previous
import functools

import jax
import jax.numpy as jnp
from jax import lax
from jax.experimental import pallas as pl
from jax.experimental.pallas import tpu as pltpu
from jax.experimental.pallas import tpu_sc as plsc

N = 10000
E = 320000
D = 128
T = 10
STEPS = 8
B = 64
L = 128
DS = 512
C = 14

NC = 2
NS = 16
NW = NC * NS
K = 128
NCHUNK = 79
PW = NCHUNK * K
E_PAD = NW * PW
NACC = N + 112
RPS = NACC // NS

BN = 2000
NBLK = N // BN
BN1 = 1000
NBLK1 = N // BN1
BL = 32


def _build_sc_agg():
    mesh = plsc.VectorSubcoreMesh(core_axis_name="c", subcore_axis_name="s")

    @functools.partial(
        pl.kernel,
        mesh=mesh,
        out_type=jax.ShapeDtypeStruct((NC, NACC, D), jnp.float32),
        scratch_types=[
            pltpu.VMEM_SHARED((NACC, D), jnp.float32),
            pltpu.VMEM((K,), jnp.int32),
            pltpu.VMEM((K,), jnp.int32),
            pltpu.VMEM((K, D), jnp.float32),
            pltpu.SemaphoreType.DMA,
        ],
    )
    def sc_agg(trans_hbm, gidx_hbm, dst_hbm, zeros_hbm, out_hbm,
               acc, idxc, dstc, rows, sem):
        c = lax.axis_index("c")
        s = lax.axis_index("s")
        wid = s * NC + c
        row0 = s * RPS
        pltpu.sync_copy(zeros_hbm.at[pl.ds(row0, RPS)],
                        acc.at[pl.ds(row0, RPS)])
        plsc.subcore_barrier()
        base = wid * PW

        def body(j, carry):
            off = base + j * K
            pltpu.sync_copy(gidx_hbm.at[pl.ds(off, K)], idxc)
            pltpu.sync_copy(dst_hbm.at[pl.ds(off, K)], dstc)
            pltpu.async_copy(trans_hbm.at[idxc], rows, sem).wait()
            pltpu.sync_copy(rows, acc.at[dstc], add=True)
            return carry

        lax.fori_loop(0, NCHUNK, body, 0)
        plsc.subcore_barrier()
        pltpu.sync_copy(acc.at[pl.ds(row0, RPS)],
                        out_hbm.at[c, pl.ds(row0, RPS)])

    return sc_agg


def _transform_body(h_ref, w_ref, b_ref, out_ref):
    out_ref[0] = (
        jnp.dot(h_ref[...], w_ref[0], preferred_element_type=jnp.float32)
        + b_ref[0]
    )


def _build_transform(interpret=False):
    return pl.pallas_call(
        _transform_body,
        grid=(T, NBLK),
        in_specs=[
            pl.BlockSpec((BN, D), lambda t, n: (n, 0)),
            pl.BlockSpec((1, D, D), lambda t, n: (t, 0, 0)),
            pl.BlockSpec((1, 1, D), lambda t, n: (t, 0, 0)),
        ],
        out_specs=pl.BlockSpec((1, BN, D), lambda t, n: (t, n, 0)),
        out_shape=jax.ShapeDtypeStruct((T, N, D), jnp.float32),
        interpret=interpret,
    )


def _gru_body(part_ref, h_ref, wih_ref, whh_ref, bih_ref, bhh_ref, out_ref):
    agg = part_ref[0] + part_ref[1]
    h = h_ref[...]
    gi = jnp.dot(agg, wih_ref[...], preferred_element_type=jnp.float32) + bih_ref[...]
    gh = jnp.dot(h, whh_ref[...], preferred_element_type=jnp.float32) + bhh_ref[...]
    r = jax.nn.sigmoid(gi[:, 0:D] + gh[:, 0:D])
    z = jax.nn.sigmoid(gi[:, D:2 * D] + gh[:, D:2 * D])
    n = jnp.tanh(gi[:, 2 * D:3 * D] + r * gh[:, 2 * D:3 * D])
    out_ref[...] = (1.0 - z) * n + z * h


def _build_gru(interpret=False):
    return pl.pallas_call(
        _gru_body,
        grid=(NBLK,),
        in_specs=[
            pl.BlockSpec((NC, BN, D), lambda n: (0, n, 0)),
            pl.BlockSpec((BN, D), lambda n: (n, 0)),
            pl.BlockSpec((D, 3 * D), lambda n: (0, 0)),
            pl.BlockSpec((D, 3 * D), lambda n: (0, 0)),
            pl.BlockSpec((1, 3 * D), lambda n: (0, 0)),
            pl.BlockSpec((1, 3 * D), lambda n: (0, 0)),
        ],
        out_specs=pl.BlockSpec((BN, D), lambda n: (n, 0)),
        out_shape=jax.ShapeDtypeStruct((N, D), jnp.float32),
        interpret=interpret,
    )


def _gru_tr_body(part_ref, h_ref, wih_ref, whh_ref, bih_ref, bhh_ref,
                 we_ref, be_ref, out_ref, tr_ref):
    agg = part_ref[0] + part_ref[1]
    h = h_ref[...]
    gi = jnp.dot(agg, wih_ref[...], preferred_element_type=jnp.float32) + bih_ref[...]
    gh = jnp.dot(h, whh_ref[...], preferred_element_type=jnp.float32) + bhh_ref[...]
    r = jax.nn.sigmoid(gi[:, 0:D] + gh[:, 0:D])
    z = jax.nn.sigmoid(gi[:, D:2 * D] + gh[:, D:2 * D])
    n = jnp.tanh(gi[:, 2 * D:3 * D] + r * gh[:, 2 * D:3 * D])
    hn = (1.0 - z) * n + z * h
    out_ref[...] = hn
    for t in range(T):
        tr_ref[t] = (
            jnp.dot(hn, we_ref[t], preferred_element_type=jnp.float32)
            + be_ref[t]
        )


def _build_gru_tr(interpret=False):
    return pl.pallas_call(
        _gru_tr_body,
        grid=(NBLK,),
        in_specs=[
            pl.BlockSpec((NC, BN, D), lambda n: (0, n, 0)),
            pl.BlockSpec((BN, D), lambda n: (n, 0)),
            pl.BlockSpec((D, 3 * D), lambda n: (0, 0)),
            pl.BlockSpec((D, 3 * D), lambda n: (0, 0)),
            pl.BlockSpec((1, 3 * D), lambda n: (0, 0)),
            pl.BlockSpec((1, 3 * D), lambda n: (0, 0)),
            pl.BlockSpec((T, D, D), lambda n: (0, 0, 0)),
            pl.BlockSpec((T, 1, D), lambda n: (0, 0, 0)),
        ],
        out_specs=[
            pl.BlockSpec((BN, D), lambda n: (n, 0)),
            pl.BlockSpec((T, BN, D), lambda n: (0, n, 0)),
        ],
        out_shape=[
            jax.ShapeDtypeStruct((N, D), jnp.float32),
            jax.ShapeDtypeStruct((T, N, D), jnp.float32),
        ],
        interpret=interpret,
    )


def _gsum_body(h_ref, x_ref, gid_ref, out_ref):
    i = pl.program_id(0)
    gids = gid_ref[0, 0, :]
    b_iota = lax.broadcasted_iota(jnp.int32, (B, BN1), 0)
    onehot = (gids[None, :] == b_iota).astype(jnp.float32)
    feat = jnp.concatenate(
        [h_ref[...], x_ref[...], jnp.ones((BN1, D), jnp.float32)], axis=1)
    contrib = jnp.dot(onehot, feat, preferred_element_type=jnp.float32)

    @pl.when(i == 0)
    def _():
        out_ref[...] = jnp.zeros_like(out_ref)

    out_ref[...] += contrib


def _build_gsum(interpret=False):
    return pl.pallas_call(
        _gsum_body,
        grid=(NBLK1,),
        in_specs=[
            pl.BlockSpec((BN1, D), lambda n: (n, 0)),
            pl.BlockSpec((BN1, D), lambda n: (n, 0)),
            pl.BlockSpec((1, 1, BN1), lambda n: (n, 0, 0)),
        ],
        out_specs=pl.BlockSpec((B, 3 * D), lambda n: (0, 0)),
        out_shape=jax.ShapeDtypeStruct((B, 3 * D), jnp.float32),
        interpret=interpret,
    )


def _seqsum_body(seq_ref, out_ref):
    i = pl.program_id(0)

    @pl.when(i == 0)
    def _():
        out_ref[...] = jnp.zeros_like(out_ref)

    out_ref[...] += jnp.sum(seq_ref[...], axis=1)


def _build_seqsum(interpret=False):
    return pl.pallas_call(
        _seqsum_body,
        grid=(L // BL,),
        in_specs=[pl.BlockSpec((B, BL, DS), lambda i: (0, i, 0))],
        out_specs=pl.BlockSpec((B, DS), lambda i: (0, 0)),
        out_shape=jax.ShapeDtypeStruct((B, DS), jnp.float32),
        interpret=interpret,
    )


def _final_body(gsum_ref, ssum_ref, wseq, bseq, wg0, bg0, wg1, bg1, wg2, bg2,
                ws0, bs0, ws1, bs1, ws2, bs2, out_ref):
    gs = gsum_ref[...]
    sums = gs[:, 0:2 * D]
    cnt = gs[:, 2 * D:2 * D + 1]
    gfeat = sums / jnp.maximum(cnt, 1.0)
    f = jnp.dot

    def relu(v):
        return jnp.maximum(v, 0.0)

    h1 = relu(f(gfeat, wg0[...], preferred_element_type=jnp.float32) + bg0[...])
    h2 = relu(f(h1, wg1[...], preferred_element_type=jnp.float32) + bg1[...])
    l1 = f(h2, wg2[...], preferred_element_type=jnp.float32) + bg2[...]
    smean = ssum_ref[...] * (1.0 / L)
    sf = relu(f(smean, wseq[...], preferred_element_type=jnp.float32) + bseq[...])
    s1 = relu(f(sf, ws0[...], preferred_element_type=jnp.float32) + bs0[...])
    s2 = relu(f(s1, ws1[...], preferred_element_type=jnp.float32) + bs1[...])
    l2 = f(s2, ws2[...], preferred_element_type=jnp.float32) + bs2[...]
    out_ref[...] = l1 + l2


def _build_final(interpret=False):
    full = lambda *shape: pl.BlockSpec(shape, lambda: (0,) * len(shape))
    return pl.pallas_call(
        _final_body,
        grid=(),
        in_specs=[
            full(B, 3 * D),
            full(B, DS),
            full(DS, 1024), full(1, 1024),
            full(2 * D, 128), full(1, 128),
            full(128, 64), full(1, 64),
            full(64, C), full(1, C),
            full(1024, 512), full(1, 512),
            full(512, 256), full(1, 256),
            full(256, C), full(1, C),
        ],
        out_specs=full(B, C),
        out_shape=jax.ShapeDtypeStruct((B, C), jnp.float32),
        interpret=interpret,
    )


def kernel(x, edge_index, edge_type, graph_ids, sequences,
           W_edge, b_edge, W_ih, W_hh, b_ih, b_hh, W_seq, b_seq,
           Wg0, bg0, Wg1, bg1, Wg2, bg2,
           Ws0, bs0, Ws1, bs1, Ws2, bs2):
    src = edge_index[0].astype(jnp.int32)
    dst = edge_index[1].astype(jnp.int32)
    et = edge_type.astype(jnp.int32)
    gidx = et * N + src
    pad = E_PAD - E
    gidx_p = jnp.concatenate([gidx, jnp.zeros((pad,), jnp.int32)])
    dst_p = jnp.concatenate([dst, jnp.full((pad,), N, jnp.int32)])
    zeros = jnp.zeros((NACC, D), jnp.float32)

    gids3 = graph_ids.astype(jnp.int32).reshape(NBLK1, 1, BN1)
    bih2 = b_ih.reshape(1, 3 * D)
    bhh2 = b_hh.reshape(1, 3 * D)

    transform = _build_transform()
    sc_agg = _build_sc_agg()
    gru = _build_gru()
    gru_tr = _build_gru_tr()
    gsum_k = _build_gsum()
    seqsum_k = _build_seqsum()
    final_k = _build_final()

    be3 = b_edge.reshape(T, 1, D)
    h = x
    trans = transform(h, W_edge, be3).reshape(T * N, D)
    for step in range(STEPS):
        parts = sc_agg(trans, gidx_p, dst_p, zeros)
        if step < STEPS - 1:
            h, trans = gru_tr(parts, h, W_ih, W_hh, bih2, bhh2, W_edge, be3)
            trans = trans.reshape(T * N, D)
        else:
            h = gru(parts, h, W_ih, W_hh, bih2, bhh2)

    gsum = gsum_k(h, x, gids3)
    ssum = seqsum_k(sequences)
    return final_k(
        gsum, ssum,
        W_seq, b_seq.reshape(1, 1024),
        Wg0, bg0.reshape(1, 128), Wg1, bg1.reshape(1, 64),
        Wg2, bg2.reshape(1, C),
        Ws0, bs0.reshape(1, 512), Ws1, bs1.reshape(1, 256),
        Ws2, bs2.reshape(1, C),
    )
